# Initial kernel scaffold; baseline (speedup 1.0000x reference)
import functools

import jax
import jax.numpy as jnp
from jax import lax
from jax.experimental import pallas as pl
from jax.experimental.pallas import tpu as pltpu
from jax.experimental.pallas import tpu_sc as plsc

K = 1024
D = 768
N = 32768
MOM = 0.9
NC, NS = 2, 16
NW = NC * NS
TOK_PER_W = N // NW
CHUNK = 128
NCHUNK = TOK_PER_W // CHUNK
ROWS_PER_TILE = K // NS


def _norm_body(x_ref, o_ref):
    x = x_ref[...]
    ss = jnp.sum(x * x, axis=1, keepdims=True)
    o_ref[...] = x * (1.0 / jnp.maximum(jnp.sqrt(ss), 1e-12))


def _normalize(x):
    blk = 2048
    return pl.pallas_call(
        _norm_body,
        out_shape=jax.ShapeDtypeStruct((N, D), jnp.float32),
        grid=(N // blk,),
        in_specs=[pl.BlockSpec((blk, D), lambda i: (i, 0))],
        out_specs=pl.BlockSpec((blk, D), lambda i: (i, 0)),
    )(x)


@functools.partial(
    pl.kernel,
    out_type=[
        jax.ShapeDtypeStruct((NC, K, D), jnp.float32),
        jax.ShapeDtypeStruct((NC, K, 16), jnp.float32),
    ],
    mesh=plsc.VectorSubcoreMesh(core_axis_name="c", subcore_axis_name="s"),
    scratch_types=[
        pltpu.VMEM((CHUNK, D), jnp.float32),
        pltpu.VMEM((CHUNK,), jnp.int32),
        pltpu.VMEM((CHUNK, 16), jnp.float32),
        pltpu.VMEM((ROWS_PER_TILE, 16), jnp.float32),
        pltpu.VMEM_SHARED((K, D), jnp.float32),
        pltpu.VMEM_SHARED((K, 16), jnp.float32),
    ],
)
def _sc_segment(p_hbm, lab_hbm, sums_out, cnt_out,
                rows_v, lab_v, ones_v, z16_v, sums_sh, cnt_sh):
    c = lax.axis_index("c")
    s = lax.axis_index("s")
    wid = c * NS + s

    zero16 = jnp.zeros((16,), jnp.float32)
    one16 = jnp.ones((16,), jnp.float32)

    def _fill(r, _):
        for j in range(D // 16):
            rows_v[r, pl.ds(j * 16, 16)] = zero16
        return 0
    lax.fori_loop(0, ROWS_PER_TILE, _fill, 0)

    def _fill_small(r, _):
        ones_v[r, :] = one16
        return 0
    lax.fori_loop(0, CHUNK, _fill_small, 0)

    def _fill_z16(r, _):
        z16_v[r, :] = zero16
        return 0
    lax.fori_loop(0, ROWS_PER_TILE, _fill_z16, 0)

    row0 = s * ROWS_PER_TILE
    pltpu.sync_copy(rows_v.at[pl.ds(0, ROWS_PER_TILE)],
                    sums_sh.at[pl.ds(row0, ROWS_PER_TILE)])
    pltpu.sync_copy(z16_v, cnt_sh.at[pl.ds(row0, ROWS_PER_TILE)])
    plsc.subcore_barrier()

    for k in range(NCHUNK):
        base = wid * TOK_PER_W + k * CHUNK
        pltpu.sync_copy(p_hbm.at[pl.ds(base, CHUNK)], rows_v)
        pltpu.sync_copy(lab_hbm.at[pl.ds(base, CHUNK)], lab_v)
        pltpu.sync_copy(rows_v, sums_sh.at[lab_v], add=True)
        pltpu.sync_copy(ones_v, cnt_sh.at[lab_v], add=True)

    plsc.subcore_barrier()

    pltpu.sync_copy(sums_sh.at[pl.ds(row0, ROWS_PER_TILE)],
                    sums_out.at[c, pl.ds(row0, ROWS_PER_TILE)])
    pltpu.sync_copy(cnt_sh.at[pl.ds(row0, ROWS_PER_TILE)],
                    cnt_out.at[c, pl.ds(row0, ROWS_PER_TILE)])


def _finish_body(s0_ref, s1_ref, c0_ref, c1_ref, cb_ref, cnt_ref, ocb_ref, ocnt_ref):
    sums = s0_ref[...] + s1_ref[...]
    wsum = c0_ref[...][:, 0:1] + c1_ref[...][:, 0:1]
    present = wsum > 0.0
    proto = sums / jnp.maximum(wsum, 1e-6)
    pn = jnp.sqrt(jnp.sum(proto * proto, axis=1, keepdims=True))
    proto = proto / jnp.maximum(pn, 1e-12)
    cb = cb_ref[...]
    ema = MOM * cb + (1.0 - MOM) * proto
    en = jnp.sqrt(jnp.sum(ema * ema, axis=1, keepdims=True))
    ema = ema / jnp.maximum(en, 1e-12)
    cnt = cnt_ref[...]
    first = cnt == 0
    new = jnp.where(first, proto, ema)
    ocb_ref[...] = jnp.where(present, new, cb)
    ocnt_ref[...] = cnt + present.astype(jnp.int32)


def _finish(s0, s1, c0, c1, cb, cnt):
    return pl.pallas_call(
        _finish_body,
        out_shape=[
            jax.ShapeDtypeStruct((K, D), jnp.float32),
            jax.ShapeDtypeStruct((K, 1), jnp.int32),
        ],
    )(s0, s1, c0, c1, cb, cnt)


def kernel(patch_proj, patch_labels, prototype_codebook, prototype_counts):
    p = _normalize(patch_proj)
    sums_p, cnt_p = _sc_segment(p, patch_labels.astype(jnp.int32))
    cb, cnt = _finish(sums_p[0], sums_p[1], cnt_p[0], cnt_p[1],
                      prototype_codebook, prototype_counts.reshape(K, 1))
    return cb, cnt.reshape(K)


# speedup vs baseline: 1.1156x; 1.1156x over previous
import functools

import jax
import jax.numpy as jnp
from jax import lax
from jax.experimental import pallas as pl
from jax.experimental.pallas import tpu as pltpu
from jax.experimental.pallas import tpu_sc as plsc

K = 1024
D = 768
N = 32768
MOM = 0.9
NC, NS = 2, 16
COLS = D // NS
TOK_PER_C = N // NC
IDXW = 128
BIG = 512
NROUND = TOK_PER_C // BIG


def _norm_body(x_ref, o_ref):
    x = x_ref[...]
    ss = jnp.sum(x * x, axis=1, keepdims=True)
    o_ref[...] = x * (1.0 / jnp.maximum(jnp.sqrt(ss), 1e-12))


def _normalize(x):
    blk = 2048
    return pl.pallas_call(
        _norm_body,
        out_shape=jax.ShapeDtypeStruct((N, D), jnp.float32),
        grid=(N // blk,),
        in_specs=[pl.BlockSpec((blk, D), lambda i: (i, 0))],
        out_specs=pl.BlockSpec((blk, D), lambda i: (i, 0)),
    )(x)


@functools.partial(
    pl.kernel,
    out_type=[
        jax.ShapeDtypeStruct((NC, K, D), jnp.float32),
        jax.ShapeDtypeStruct((NC, K, 16), jnp.float32),
    ],
    mesh=plsc.VectorSubcoreMesh(core_axis_name="c", subcore_axis_name="s"),
    compiler_params=pltpu.CompilerParams(use_tc_tiling_on_sc=False,
                                         needs_layout_passes=False),
    scratch_types=[
        pltpu.VMEM((BIG, COLS), jnp.float32),
        pltpu.VMEM((BIG // IDXW, IDXW), jnp.int32),
        pltpu.VMEM((K, COLS), jnp.float32),
        pltpu.VMEM((K, 16), jnp.float32),
    ],
)
def _sc_segment(p_hbm, lab_hbm, sums_out, cnt_out,
                rows_v, lab_v, acc_v, cnt_v):
    c = lax.axis_index("c")
    s = lax.axis_index("s")
    col0 = s * COLS
    ii = lax.iota(jnp.int32, 16)

    zero16 = jnp.zeros((16,), jnp.float32)
    one16 = jnp.ones((16,), jnp.float32)

    def _zero_acc(r, _):
        for j in range(COLS // 16):
            acc_v[r, pl.ds(j * 16, 16)] = zero16
        cnt_v[r, :] = zero16
        return 0
    lax.fori_loop(0, K, _zero_acc, 0)

    def _round(i, _):
        base = c * TOK_PER_C + i * BIG
        pltpu.sync_copy(p_hbm.at[pl.ds(base, BIG), pl.ds(col0, COLS)], rows_v)
        pltpu.sync_copy(lab_hbm.at[pl.ds(base // IDXW, BIG // IDXW)], lab_v)

        def _group(g, _):
            lv = lab_v[g // (IDXW // 16), pl.ds((g % (IDXW // 16)) * 16, 16)]
            for k in range(16):
                row = jnp.full((16,), lv[k], jnp.int32)
                t = g * 16 + k
                for j in range(COLS // 16):
                    x = rows_v[t, pl.ds(j * 16, 16)]
                    plsc.addupdate_scatter(acc_v, [row, ii + (j * 16)], x)

            @pl.when(s == 0)
            def _():
                plsc.addupdate_scatter(cnt_v, [lv, ii], one16)
            return 0
        lax.fori_loop(0, BIG // 16, _group, 0)
        return 0
    lax.fori_loop(0, NROUND, _round, 0)

    pltpu.sync_copy(acc_v, sums_out.at[c, :, pl.ds(col0, COLS)])

    @pl.when(s == 0)
    def _():
        pltpu.sync_copy(cnt_v, cnt_out.at[c])


def _finish_body(s0_ref, s1_ref, c0_ref, c1_ref, cb_ref, cnt_ref, ocb_ref, ocnt_ref):
    sums = s0_ref[...] + s1_ref[...]
    wsum = jnp.sum(c0_ref[...] + c1_ref[...], axis=1, keepdims=True)
    present = wsum > 0.0
    proto = sums / jnp.maximum(wsum, 1e-6)
    pn = jnp.sqrt(jnp.sum(proto * proto, axis=1, keepdims=True))
    proto = proto / jnp.maximum(pn, 1e-12)
    cb = cb_ref[...]
    ema = MOM * cb + (1.0 - MOM) * proto
    en = jnp.sqrt(jnp.sum(ema * ema, axis=1, keepdims=True))
    ema = ema / jnp.maximum(en, 1e-12)
    cnt = cnt_ref[...]
    first = cnt == 0
    new = jnp.where(first, proto, ema)
    ocb_ref[...] = jnp.where(present, new, cb)
    ocnt_ref[...] = cnt + present.astype(jnp.int32)


def _finish(s0, s1, c0, c1, cb, cnt):
    return pl.pallas_call(
        _finish_body,
        out_shape=[
            jax.ShapeDtypeStruct((K, D), jnp.float32),
            jax.ShapeDtypeStruct((K, 1), jnp.int32),
        ],
    )(s0, s1, c0, c1, cb, cnt)


def kernel(patch_proj, patch_labels, prototype_codebook, prototype_counts):
    p = _normalize(patch_proj)
    lab2d = patch_labels.astype(jnp.int32).reshape(N // IDXW, IDXW)
    sums_p, cnt_p = _sc_segment(p, lab2d)
    cb, cnt = _finish(sums_p[0], sums_p[1], cnt_p[0], cnt_p[1],
                      prototype_codebook, prototype_counts.reshape(K, 1))
    return cb, cnt.reshape(K)
